# baseline (speedup 1.0000x reference)
import functools
import math

import numpy as np
import jax
import jax.numpy as jnp
from jax import lax
from jax.experimental import pallas as pl
from jax.experimental.pallas import tpu as pltpu
from jax.experimental.pallas import tpu_sc as plsc

_B, _L, _H, _D = 2, 2048, 12, 64
_BH = _B * _H
_U = 40
_QBLK = 256
_NQB = _L // _QBLK
_SCALE = 1.0 / math.sqrt(_D)
_NEG = -3.0e38


def _rotl32(x, r):
    return ((x << np.uint32(r)) | (x >> np.uint32(32 - r))).astype(np.uint32)


def _threefry2x32(k0, k1, x0, x1):
    ks = [np.uint32(k0), np.uint32(k1), np.uint32(k0 ^ k1 ^ 0x1BD11BDA)]
    rot = [(13, 15, 26, 6), (17, 29, 16, 24)]
    x0 = (x0 + ks[0]).astype(np.uint32)
    x1 = (x1 + ks[1]).astype(np.uint32)
    for i in range(5):
        for r in rot[i % 2]:
            x0 = (x0 + x1).astype(np.uint32)
            x1 = _rotl32(x1, r)
            x1 = x1 ^ x0
        x0 = (x0 + ks[(i + 1) % 3]).astype(np.uint32)
        x1 = (x1 + ks[(i + 2) % 3] + np.uint32(i + 1)).astype(np.uint32)
    return x0, x1


def _build_cnt() -> np.ndarray:
    k2 = (np.uint32(64467757), np.uint32(2916123636))
    n = _L * _U
    i = np.arange(n, dtype=np.uint32)
    y0, y1 = _threefry2x32(k2[0], k2[1], np.zeros(n, np.uint32), i)
    idx = ((y0 ^ y1) % np.uint32(_L)).astype(np.int64).reshape(_L, _U)
    cnt = np.zeros((_L, _L), np.float32)
    np.add.at(cnt, (np.arange(_L)[:, None], idx), 1.0)
    return cnt


_CNT_NP = _build_cnt()


_UP = 48


def _m_body(q_ref, k_ref, cnt_ref, m_ref):
    k = k_ref[0]

    m_rows = []
    for qi in range(_NQB):
        qblk = q_ref[0, qi * _QBLK:(qi + 1) * _QBLK, :]
        s = lax.dot_general(qblk, k, (((1,), (1,)), ((), ())),
                            preferred_element_type=jnp.float32)
        cnt = cnt_ref[qi * _QBLK:(qi + 1) * _QBLK, :]
        smax = jnp.max(jnp.where(cnt > 0.0, s, _NEG), axis=1)
        ssum = jnp.sum(s * cnt, axis=1)
        m_rows.append(smax - ssum * (1.0 / _L))
    m_ref[0] = jnp.stack(m_rows, axis=0)


_GDNUMS = jax.lax.GatherDimensionNumbers(
    offset_dims=(), collapsed_slice_dims=(0,), start_index_map=(0,))


def _lane_shuffle(v, idx16):
    return lax.gather(v, idx16.reshape(16, 1), _GDNUMS, (1,),
                      mode=lax.GatherScatterMode.PROMISE_IN_BOUNDS)


def _sc_topk_body(m_hbm, idx_hbm, m_vmem, idx_vmem):
    nc = 2
    wid = lax.axis_index("s") * nc + lax.axis_index("c")

    @pl.when(wid < _BH)
    def _():
        pltpu.sync_copy(m_hbm.at[wid], m_vmem)
        iota = lax.iota(jnp.int32, 16)
        big_i = jnp.zeros((16,), jnp.int32) + _L
        neg1 = jnp.zeros((16,), jnp.int32) - 1
        pos_f = jnp.zeros((16,), jnp.float32) + 3.0e38
        neg_f = jnp.zeros((16,), jnp.float32) + _NEG

        def body(i, carry):
            m_prev, t_prev, idx0, idx1, idx2 = carry
            best_v, best_i = neg_f, big_i
            for c in range(_L // 16):
                vv = m_vmem[pl.ds(16 * c, 16)]
                gidx = iota + (16 * c)
                elig = (vv < m_prev) | ((vv == m_prev) & (gidx > t_prev))
                vv = jnp.where(elig, vv, _NEG)
                better = (vv > best_v) | ((vv == best_v) & (gidx < best_i))
                best_v = jnp.where(better, vv, best_v)
                best_i = jnp.where(better, gidx, best_i)
            for k in (1, 2, 4, 8):
                ov = _lane_shuffle(best_v, jnp.bitwise_xor(iota, k))
                oi = _lane_shuffle(best_i, jnp.bitwise_xor(iota, k))
                better = (ov > best_v) | ((ov == best_v) & (oi < best_i))
                best_v = jnp.where(better, ov, best_v)
                best_i = jnp.where(better, oi, best_i)
            idx0 = jnp.where(iota == i, best_i, idx0)
            idx1 = jnp.where(iota + 16 == i, best_i, idx1)
            idx2 = jnp.where(iota + 32 == i, best_i, idx2)
            return best_v, best_i, idx0, idx1, idx2

        carry = lax.fori_loop(0, _U, body, (pos_f, neg1, neg1, neg1, neg1))
        idx_vmem[pl.ds(0, 16)] = carry[2]
        idx_vmem[pl.ds(16, 16)] = carry[3]
        idx_vmem[pl.ds(32, 16)] = carry[4]
        pltpu.sync_copy(idx_vmem, idx_hbm.at[wid])


def _attn_body(idx_ref, q_ref, k_ref, v_ref, o_ref):
    idx_row = idx_ref[0]

    eye = (jax.lax.broadcasted_iota(jnp.int32, (_UP, _UP), 0)
           == jax.lax.broadcasted_iota(jnp.int32, (_UP, _UP), 1))
    idx_col = jnp.sum(jnp.where(eye, jnp.broadcast_to(idx_row, (_UP, _UP)), 0),
                      axis=1, keepdims=True)

    sel = (jax.lax.broadcasted_iota(jnp.int32, (_UP, _L), 1)
           == idx_col).astype(jnp.float32)
    sel_t = (jax.lax.broadcasted_iota(jnp.int32, (_L, _UP), 0)
             == idx_row).astype(jnp.float32)

    qr = jnp.dot(sel, q_ref[0], preferred_element_type=jnp.float32)
    scores = lax.dot_general(qr, k_ref[0], (((1,), (1,)), ((), ())),
                             preferred_element_type=jnp.float32) * _SCALE
    smax = jnp.max(scores, axis=1, keepdims=True)
    p = jnp.exp(scores - smax)
    attn = p / jnp.sum(p, axis=1, keepdims=True)
    upd = jnp.dot(attn, v_ref[0], preferred_element_type=jnp.float32)

    vmean = jnp.mean(v_ref[0], axis=0, keepdims=True)
    delta = jnp.dot(sel_t, upd - vmean, preferred_element_type=jnp.float32)
    o_ref[0] = vmean + delta


@jax.jit
def _run(q, k, v, cnt):
    m = pl.pallas_call(
        _m_body,
        grid=(_BH,),
        in_specs=[
            pl.BlockSpec((1, _L, _D), lambda i: (i, 0, 0)),
            pl.BlockSpec((1, _L, _D), lambda i: (i, 0, 0)),
            pl.BlockSpec((_L, _L), lambda i: (0, 0)),
        ],
        out_specs=pl.BlockSpec((1, _NQB, _QBLK), lambda i: (i, 0, 0)),
        out_shape=jax.ShapeDtypeStruct((_BH, _NQB, _QBLK), jnp.float32),
    )(q, k, cnt)

    sc_topk = functools.partial(
        pl.kernel,
        out_type=jax.ShapeDtypeStruct((_BH, _UP), jnp.int32),
        mesh=plsc.VectorSubcoreMesh(core_axis_name="c", subcore_axis_name="s"),
        scratch_types=[
            pltpu.VMEM((_L,), jnp.float32),
            pltpu.VMEM((_UP,), jnp.int32),
        ],
    )(_sc_topk_body)
    idx = sc_topk(m.reshape(_BH, _L))
    return pl.pallas_call(
        _attn_body,
        grid=(_BH,),
        in_specs=[
            pl.BlockSpec((1, 1, _UP), lambda i: (i, 0, 0)),
            pl.BlockSpec((1, _L, _D), lambda i: (i, 0, 0)),
            pl.BlockSpec((1, _L, _D), lambda i: (i, 0, 0)),
            pl.BlockSpec((1, _L, _D), lambda i: (i, 0, 0)),
        ],
        out_specs=pl.BlockSpec((1, _L, _D), lambda i: (i, 0, 0)),
        out_shape=jax.ShapeDtypeStruct((_BH, _L, _D), jnp.float32),
    )(idx.reshape(_BH, 1, _UP), q, k, v)


def kernel(queries, keys, values, attn_mask):
    q = jnp.transpose(queries, (0, 2, 1, 3)).reshape(_BH, _L, _D)
    k = jnp.transpose(keys, (0, 2, 1, 3)).reshape(_BH, _L, _D)
    v = jnp.transpose(values, (0, 2, 1, 3)).reshape(_BH, _L, _D)
    cnt = jnp.asarray(_CNT_NP)
    out = _run(q, k, v, cnt)
    return out.reshape(_B, _H, _L, _D)


# speedup vs baseline: 1.1057x; 1.1057x over previous
import functools
import math

import numpy as np
import jax
import jax.numpy as jnp
from jax import lax
from jax.experimental import pallas as pl
from jax.experimental.pallas import tpu as pltpu
from jax.experimental.pallas import tpu_sc as plsc

_B, _L, _H, _D = 2, 2048, 12, 64
_BH = _B * _H
_U = 40
_QBLK = 256
_NQB = _L // _QBLK
_SCALE = 1.0 / math.sqrt(_D)
_NEG = -3.0e38


def _rotl32(x, r):
    return ((x << np.uint32(r)) | (x >> np.uint32(32 - r))).astype(np.uint32)


def _threefry2x32(k0, k1, x0, x1):
    ks = [np.uint32(k0), np.uint32(k1), np.uint32(k0 ^ k1 ^ 0x1BD11BDA)]
    rot = [(13, 15, 26, 6), (17, 29, 16, 24)]
    x0 = (x0 + ks[0]).astype(np.uint32)
    x1 = (x1 + ks[1]).astype(np.uint32)
    for i in range(5):
        for r in rot[i % 2]:
            x0 = (x0 + x1).astype(np.uint32)
            x1 = _rotl32(x1, r)
            x1 = x1 ^ x0
        x0 = (x0 + ks[(i + 1) % 3]).astype(np.uint32)
        x1 = (x1 + ks[(i + 2) % 3] + np.uint32(i + 1)).astype(np.uint32)
    return x0, x1


def _build_cnt() -> np.ndarray:
    k2 = (np.uint32(64467757), np.uint32(2916123636))
    n = _L * _U
    i = np.arange(n, dtype=np.uint32)
    y0, y1 = _threefry2x32(k2[0], k2[1], np.zeros(n, np.uint32), i)
    idx = ((y0 ^ y1) % np.uint32(_L)).astype(np.int64).reshape(_L, _U)
    cnt = np.zeros((_L, _L), np.float32)
    np.add.at(cnt, (np.arange(_L)[:, None], idx), 1.0)
    return cnt


_CNT_NP = _build_cnt()
_BIAS_NP = np.where(_CNT_NP > 0, np.float32(0), np.float32(_NEG)).astype(np.float32)


_UP = 48


def _m_body(q_ref, kt_ref, cnt_ref, bias_ref, m_ref):
    kt = kt_ref[0]

    m_rows = []
    for qi in range(_NQB):
        qblk = q_ref[0, qi * _QBLK:(qi + 1) * _QBLK, :]
        s = jnp.dot(qblk, kt, preferred_element_type=jnp.float32)
        cnt = cnt_ref[qi * _QBLK:(qi + 1) * _QBLK, :]
        bias = bias_ref[qi * _QBLK:(qi + 1) * _QBLK, :]
        smax = jnp.max(s + bias, axis=1)
        ssum = jnp.sum(s * cnt, axis=1)
        m_rows.append(smax - ssum * (1.0 / _L))
    m_ref[0] = jnp.stack(m_rows, axis=0)


_GDNUMS = jax.lax.GatherDimensionNumbers(
    offset_dims=(), collapsed_slice_dims=(0,), start_index_map=(0,))


def _lane_shuffle(v, idx16):
    return lax.gather(v, idx16.reshape(16, 1), _GDNUMS, (1,),
                      mode=lax.GatherScatterMode.PROMISE_IN_BOUNDS)


def _sc_topk_body(m_hbm, idx_hbm, m_vmem, idx_vmem):
    nc = 2
    wid = lax.axis_index("s") * nc + lax.axis_index("c")

    @pl.when(wid < _BH)
    def _():
        pltpu.sync_copy(m_hbm.at[wid], m_vmem)
        iota = lax.iota(jnp.int32, 16)
        big_i = jnp.zeros((16,), jnp.int32) + _L
        neg1 = jnp.zeros((16,), jnp.int32) - 1
        pos_f = jnp.zeros((16,), jnp.float32) + 3.0e38
        neg_f = jnp.zeros((16,), jnp.float32) + _NEG

        def body(i, carry):
            m_prev, t_prev, idx0, idx1, idx2 = carry
            best_v, best_i = neg_f, big_i
            for c in range(_L // 16):
                vv = m_vmem[pl.ds(16 * c, 16)]
                gidx = iota + (16 * c)
                elig = (vv < m_prev) | ((vv == m_prev) & (gidx > t_prev))
                vv = jnp.where(elig, vv, _NEG)
                better = (vv > best_v) | ((vv == best_v) & (gidx < best_i))
                best_v = jnp.where(better, vv, best_v)
                best_i = jnp.where(better, gidx, best_i)
            for k in (1, 2, 4, 8):
                ov = _lane_shuffle(best_v, jnp.bitwise_xor(iota, k))
                oi = _lane_shuffle(best_i, jnp.bitwise_xor(iota, k))
                better = (ov > best_v) | ((ov == best_v) & (oi < best_i))
                best_v = jnp.where(better, ov, best_v)
                best_i = jnp.where(better, oi, best_i)
            idx0 = jnp.where(iota == i, best_i, idx0)
            idx1 = jnp.where(iota + 16 == i, best_i, idx1)
            idx2 = jnp.where(iota + 32 == i, best_i, idx2)
            return best_v, best_i, idx0, idx1, idx2

        carry = lax.fori_loop(0, _U, body, (pos_f, neg1, neg1, neg1, neg1))
        idx_vmem[pl.ds(0, 16)] = carry[2]
        idx_vmem[pl.ds(16, 16)] = carry[3]
        idx_vmem[pl.ds(32, 16)] = carry[4]
        pltpu.sync_copy(idx_vmem, idx_hbm.at[wid])


def _attn_body(idx_ref, q_ref, kt_ref, v_ref, o_ref):
    kt = kt_ref[0]
    idx_row = idx_ref[0]

    eye = (jax.lax.broadcasted_iota(jnp.int32, (_UP, _UP), 0)
           == jax.lax.broadcasted_iota(jnp.int32, (_UP, _UP), 1))
    idx_col = jnp.sum(jnp.where(eye, jnp.broadcast_to(idx_row, (_UP, _UP)), 0),
                      axis=1, keepdims=True)

    sel = (jax.lax.broadcasted_iota(jnp.int32, (_UP, _L), 1)
           == idx_col).astype(jnp.float32)
    sel_t = (jax.lax.broadcasted_iota(jnp.int32, (_L, _UP), 0)
             == idx_row).astype(jnp.float32)

    qr = jnp.dot(sel, q_ref[0], preferred_element_type=jnp.float32)
    scores = jnp.dot(qr, kt, preferred_element_type=jnp.float32) * _SCALE
    smax = jnp.max(scores, axis=1, keepdims=True)
    p = jnp.exp(scores - smax)
    attn = p / jnp.sum(p, axis=1, keepdims=True)
    upd = jnp.dot(attn, v_ref[0], preferred_element_type=jnp.float32)

    vmean = jnp.mean(v_ref[0], axis=0, keepdims=True)
    delta = jnp.dot(sel_t, upd - vmean, preferred_element_type=jnp.float32)
    o_ref[0] = vmean + delta


@jax.jit
def _run(q, kt, v, cnt, bias):
    m = pl.pallas_call(
        _m_body,
        grid=(_BH,),
        in_specs=[
            pl.BlockSpec((1, _L, _D), lambda i: (i, 0, 0)),
            pl.BlockSpec((1, _D, _L), lambda i: (i, 0, 0)),
            pl.BlockSpec((_L, _L), lambda i: (0, 0)),
            pl.BlockSpec((_L, _L), lambda i: (0, 0)),
        ],
        out_specs=pl.BlockSpec((1, _NQB, _QBLK), lambda i: (i, 0, 0)),
        out_shape=jax.ShapeDtypeStruct((_BH, _NQB, _QBLK), jnp.float32),
    )(q, kt, cnt, bias)

    sc_topk = functools.partial(
        pl.kernel,
        out_type=jax.ShapeDtypeStruct((_BH, _UP), jnp.int32),
        mesh=plsc.VectorSubcoreMesh(core_axis_name="c", subcore_axis_name="s"),
        scratch_types=[
            pltpu.VMEM((_L,), jnp.float32),
            pltpu.VMEM((_UP,), jnp.int32),
        ],
    )(_sc_topk_body)
    idx = sc_topk(m.reshape(_BH, _L))
    return pl.pallas_call(
        _attn_body,
        grid=(_BH,),
        in_specs=[
            pl.BlockSpec((1, 1, _UP), lambda i: (i, 0, 0)),
            pl.BlockSpec((1, _L, _D), lambda i: (i, 0, 0)),
            pl.BlockSpec((1, _D, _L), lambda i: (i, 0, 0)),
            pl.BlockSpec((1, _L, _D), lambda i: (i, 0, 0)),
        ],
        out_specs=pl.BlockSpec((1, _L, _D), lambda i: (i, 0, 0)),
        out_shape=jax.ShapeDtypeStruct((_BH, _L, _D), jnp.float32),
    )(idx.reshape(_BH, 1, _UP), q, kt, v)


def kernel(queries, keys, values, attn_mask):
    q = jnp.transpose(queries, (0, 2, 1, 3)).reshape(_BH, _L, _D)
    kt = jnp.transpose(keys, (0, 2, 3, 1)).reshape(_BH, _D, _L)
    v = jnp.transpose(values, (0, 2, 1, 3)).reshape(_BH, _L, _D)
    cnt = jnp.asarray(_CNT_NP)
    bias = jnp.asarray(_BIAS_NP)
    out = _run(q, kt, v, cnt, bias)
    return out.reshape(_B, _H, _L, _D)


# speedup vs baseline: 1.1074x; 1.0016x over previous
import functools
import math

import numpy as np
import jax
import jax.numpy as jnp
from jax import lax
from jax.experimental import pallas as pl
from jax.experimental.pallas import tpu as pltpu
from jax.experimental.pallas import tpu_sc as plsc

_B, _L, _H, _D = 2, 2048, 12, 64
_BH = _B * _H
_U = 40
_QBLK = 512
_NQB = _L // _QBLK
_SCALE = 1.0 / math.sqrt(_D)
_NEG = -3.0e38


def _rotl32(x, r):
    return ((x << np.uint32(r)) | (x >> np.uint32(32 - r))).astype(np.uint32)


def _threefry2x32(k0, k1, x0, x1):
    ks = [np.uint32(k0), np.uint32(k1), np.uint32(k0 ^ k1 ^ 0x1BD11BDA)]
    rot = [(13, 15, 26, 6), (17, 29, 16, 24)]
    x0 = (x0 + ks[0]).astype(np.uint32)
    x1 = (x1 + ks[1]).astype(np.uint32)
    for i in range(5):
        for r in rot[i % 2]:
            x0 = (x0 + x1).astype(np.uint32)
            x1 = _rotl32(x1, r)
            x1 = x1 ^ x0
        x0 = (x0 + ks[(i + 1) % 3]).astype(np.uint32)
        x1 = (x1 + ks[(i + 2) % 3] + np.uint32(i + 1)).astype(np.uint32)
    return x0, x1


def _build_cnt() -> np.ndarray:
    k2 = (np.uint32(64467757), np.uint32(2916123636))
    n = _L * _U
    i = np.arange(n, dtype=np.uint32)
    y0, y1 = _threefry2x32(k2[0], k2[1], np.zeros(n, np.uint32), i)
    idx = ((y0 ^ y1) % np.uint32(_L)).astype(np.int64).reshape(_L, _U)
    cnt = np.zeros((_L, _L), np.float32)
    np.add.at(cnt, (np.arange(_L)[:, None], idx), 1.0)
    return cnt


_CNT_NP = _build_cnt()
_BIAS_NP = np.where(_CNT_NP > 0, np.float32(0), np.float32(_NEG)).astype(np.float32)


_UP = 48


def _m_body(q_ref, kt_ref, cnt_ref, bias_ref, m_ref):
    kt = kt_ref[0]

    m_rows = []
    for qi in range(_NQB):
        qblk = q_ref[0, qi * _QBLK:(qi + 1) * _QBLK, :]
        s = jnp.dot(qblk, kt, preferred_element_type=jnp.float32)
        cnt = cnt_ref[qi * _QBLK:(qi + 1) * _QBLK, :]
        bias = bias_ref[qi * _QBLK:(qi + 1) * _QBLK, :]
        smax = jnp.max(s + bias, axis=1)
        ssum = jnp.sum(s * cnt, axis=1)
        m_rows.append(smax - ssum * (1.0 / _L))
    m_ref[0] = jnp.stack(m_rows, axis=0)


_GDNUMS = jax.lax.GatherDimensionNumbers(
    offset_dims=(), collapsed_slice_dims=(0,), start_index_map=(0,))


def _lane_shuffle(v, idx16):
    return lax.gather(v, idx16.reshape(16, 1), _GDNUMS, (1,),
                      mode=lax.GatherScatterMode.PROMISE_IN_BOUNDS)


def _sc_topk_body(m_hbm, idx_hbm, m_vmem, idx_vmem):
    nc = 2
    wid = lax.axis_index("s") * nc + lax.axis_index("c")

    @pl.when(wid < _BH)
    def _():
        pltpu.sync_copy(m_hbm.at[wid], m_vmem)
        iota = lax.iota(jnp.int32, 16)
        big_i = jnp.zeros((16,), jnp.int32) + _L
        neg1 = jnp.zeros((16,), jnp.int32) - 1
        pos_f = jnp.zeros((16,), jnp.float32) + 3.0e38
        neg_f = jnp.zeros((16,), jnp.float32) + _NEG

        def body(i, carry):
            m_prev, t_prev, idx0, idx1, idx2 = carry
            best_v, best_i = neg_f, big_i
            for c in range(_L // 16):
                vv = m_vmem[pl.ds(16 * c, 16)]
                gidx = iota + (16 * c)
                elig = (vv < m_prev) | ((vv == m_prev) & (gidx > t_prev))
                vv = jnp.where(elig, vv, _NEG)
                better = (vv > best_v) | ((vv == best_v) & (gidx < best_i))
                best_v = jnp.where(better, vv, best_v)
                best_i = jnp.where(better, gidx, best_i)
            for k in (1, 2, 4, 8):
                ov = _lane_shuffle(best_v, jnp.bitwise_xor(iota, k))
                oi = _lane_shuffle(best_i, jnp.bitwise_xor(iota, k))
                better = (ov > best_v) | ((ov == best_v) & (oi < best_i))
                best_v = jnp.where(better, ov, best_v)
                best_i = jnp.where(better, oi, best_i)
            idx0 = jnp.where(iota == i, best_i, idx0)
            idx1 = jnp.where(iota + 16 == i, best_i, idx1)
            idx2 = jnp.where(iota + 32 == i, best_i, idx2)
            return best_v, best_i, idx0, idx1, idx2

        carry = lax.fori_loop(0, _U, body, (pos_f, neg1, neg1, neg1, neg1))
        idx_vmem[pl.ds(0, 16)] = carry[2]
        idx_vmem[pl.ds(16, 16)] = carry[3]
        idx_vmem[pl.ds(32, 16)] = carry[4]
        pltpu.sync_copy(idx_vmem, idx_hbm.at[wid])


def _attn_body(idx_ref, q_ref, kt_ref, v_ref, o_ref):
    kt = kt_ref[0]
    idx_row = idx_ref[0]

    eye = (jax.lax.broadcasted_iota(jnp.int32, (_UP, _UP), 0)
           == jax.lax.broadcasted_iota(jnp.int32, (_UP, _UP), 1))
    idx_col = jnp.sum(jnp.where(eye, jnp.broadcast_to(idx_row, (_UP, _UP)), 0),
                      axis=1, keepdims=True)

    sel = (jax.lax.broadcasted_iota(jnp.int32, (_UP, _L), 1)
           == idx_col).astype(jnp.float32)
    sel_t = (jax.lax.broadcasted_iota(jnp.int32, (_L, _UP), 0)
             == idx_row).astype(jnp.float32)

    qr = jnp.dot(sel, q_ref[0], preferred_element_type=jnp.float32)
    scores = jnp.dot(qr, kt, preferred_element_type=jnp.float32) * _SCALE
    smax = jnp.max(scores, axis=1, keepdims=True)
    p = jnp.exp(scores - smax)
    attn = p / jnp.sum(p, axis=1, keepdims=True)
    upd = jnp.dot(attn, v_ref[0], preferred_element_type=jnp.float32)

    vmean = jnp.mean(v_ref[0], axis=0, keepdims=True)
    delta = jnp.dot(sel_t, upd - vmean, preferred_element_type=jnp.float32)
    o_ref[0] = vmean + delta


@jax.jit
def _run(q, kt, v, cnt, bias):
    m = pl.pallas_call(
        _m_body,
        grid=(_BH,),
        in_specs=[
            pl.BlockSpec((1, _L, _D), lambda i: (i, 0, 0)),
            pl.BlockSpec((1, _D, _L), lambda i: (i, 0, 0)),
            pl.BlockSpec((_L, _L), lambda i: (0, 0)),
            pl.BlockSpec((_L, _L), lambda i: (0, 0)),
        ],
        out_specs=pl.BlockSpec((1, _NQB, _QBLK), lambda i: (i, 0, 0)),
        out_shape=jax.ShapeDtypeStruct((_BH, _NQB, _QBLK), jnp.float32),
    )(q, kt, cnt, bias)

    sc_topk = functools.partial(
        pl.kernel,
        out_type=jax.ShapeDtypeStruct((_BH, _UP), jnp.int32),
        mesh=plsc.VectorSubcoreMesh(core_axis_name="c", subcore_axis_name="s"),
        scratch_types=[
            pltpu.VMEM((_L,), jnp.float32),
            pltpu.VMEM((_UP,), jnp.int32),
        ],
    )(_sc_topk_body)
    idx = sc_topk(m.reshape(_BH, _L))
    return pl.pallas_call(
        _attn_body,
        grid=(_BH,),
        in_specs=[
            pl.BlockSpec((1, 1, _UP), lambda i: (i, 0, 0)),
            pl.BlockSpec((1, _L, _D), lambda i: (i, 0, 0)),
            pl.BlockSpec((1, _D, _L), lambda i: (i, 0, 0)),
            pl.BlockSpec((1, _L, _D), lambda i: (i, 0, 0)),
        ],
        out_specs=pl.BlockSpec((1, _L, _D), lambda i: (i, 0, 0)),
        out_shape=jax.ShapeDtypeStruct((_BH, _L, _D), jnp.float32),
    )(idx.reshape(_BH, 1, _UP), q, kt, v)


def kernel(queries, keys, values, attn_mask):
    q = jnp.transpose(queries, (0, 2, 1, 3)).reshape(_BH, _L, _D)
    kt = jnp.transpose(keys, (0, 2, 3, 1)).reshape(_BH, _D, _L)
    v = jnp.transpose(values, (0, 2, 1, 3)).reshape(_BH, _L, _D)
    cnt = jnp.asarray(_CNT_NP)
    bias = jnp.asarray(_BIAS_NP)
    out = _run(q, kt, v, cnt, bias)
    return out.reshape(_B, _H, _L, _D)
